# baseline (speedup 1.0000x reference)
import functools

import jax
import jax.numpy as jnp
from jax import lax
from jax.experimental import pallas as pl
from jax.experimental.pallas import tpu as pltpu
from jax.experimental.pallas import tpu_sc as plsc

N = 10000
NC, NS = 2, 16
NW = NC * NS
CHUNK = 128
RPT = 632
N_ACC = NS * RPT
DEG_ROWS = 640
DEG_RPT = DEG_ROWS // NS
DEG_CH = DEG_ROWS // CHUNK

RB = 1024
GRID = -(-N // RB)


def _sc_mesh():
    return plsc.VectorSubcoreMesh(core_axis_name="c", subcore_axis_name="s")


DEGW = DEG_ROWS * 16


def _make_deg(ept, base):
    steps = ept // 16

    @functools.partial(
        pl.kernel,
        out_type=jax.ShapeDtypeStruct((NW, DEGW), jnp.float32),
        mesh=_sc_mesh(),
        compiler_params=pltpu.CompilerParams(needs_layout_passes=False),
        scratch_types=[
            pltpu.VMEM((ept,), jnp.int32),
            pltpu.VMEM((DEGW,), jnp.float32),
        ],
    )
    def deg_kernel(dst1_hbm, zeros_hbm, out_hbm, idxv, degv):
        c = lax.axis_index("c")
        s = lax.axis_index("s")
        wid = c * NS + s
        pltpu.sync_copy(zeros_hbm, degv)
        pltpu.sync_copy(dst1_hbm.at[pl.ds(base + wid * ept, ept)], idxv)
        ones = jnp.full((16,), 1.0, jnp.float32)

        def step(i, carry):
            idx = idxv[pl.ds(i * 16, 16)]
            plsc.addupdate_scatter(degv, [idx], ones)
            return carry

        lax.fori_loop(0, steps, step, 0)
        pltpu.sync_copy(degv, out_hbm.at[wid])

    return deg_kernel


G = 8
WIDE = G * CHUNK


def _make_scatter(f, nch):
    ngrp = nch // (2 * G)

    @functools.partial(
        pl.kernel,
        out_type=[jax.ShapeDtypeStruct((N_ACC, f), jnp.float32),
                  jax.ShapeDtypeStruct((N_ACC, f), jnp.float32)],
        mesh=_sc_mesh(),
        compiler_params=pltpu.CompilerParams(use_tc_tiling_on_sc=False),
        scratch_types=[
            pltpu.VMEM((nch // G, WIDE), jnp.int32),
            pltpu.VMEM((nch, CHUNK), jnp.int32),
            pltpu.VMEM((WIDE, f), jnp.float32),
            pltpu.VMEM((WIDE, f), jnp.float32),
            pltpu.VMEM_SHARED((N_ACC, f), jnp.float32),
            pltpu.VMEM_SHARED((N_ACC, f), jnp.float32),
            pltpu.SemaphoreType.DMA,
            pltpu.SemaphoreType.DMA,
            pltpu.SemaphoreType.DMA,
            pltpu.SemaphoreType.DMA,
        ],
    )
    def scatter_kernel(z_hbm, src_hbm, dst_hbm, zeros_hbm, out0_hbm, out1_hbm,
                       idxs, idxd, rows_a, rows_b, acc, zs,
                       sem_ga, sem_gb, sem_sa, sem_sb):
        c = lax.axis_index("c")
        s = lax.axis_index("s")
        wid = c * NS + s
        pltpu.sync_copy(zeros_hbm.at[pl.ds(s * RPT, RPT)],
                        acc.at[pl.ds(s * RPT, RPT)])
        pltpu.sync_copy(z_hbm.at[pl.ds(s * RPT, RPT)],
                        zs.at[pl.ds(s * RPT, RPT)])
        pltpu.sync_copy(src_hbm.at[wid], idxs)
        pltpu.sync_copy(dst_hbm.at[wid], idxd)
        plsc.subcore_barrier()

        def gath_start(j, buf, sem):
            pltpu.async_copy(zs.at[idxs.at[j // G]], buf, sem)

        def gath_wait(j, buf, sem):
            pltpu.make_async_copy(zs.at[idxs.at[j // G]], buf, sem).wait()

        def scat_start(j, t, buf, sem):
            pltpu.async_copy(buf.at[pl.ds(t * CHUNK, CHUNK)],
                             acc.at[idxd.at[j + t]], sem, add=True)

        def scat_wait(j, t, buf, sem):
            pltpu.make_async_copy(buf.at[pl.ds(t * CHUNK, CHUNK)],
                                  acc.at[idxd.at[j + t]], sem).wait()

        gath_start(0, rows_a, sem_ga)

        def grp(k, carry):
            ja = k * 2 * G
            jb = ja + G
            gath_wait(ja, rows_a, sem_ga)

            @pl.when(k > 0)
            def _():
                for t in range(G):
                    scat_wait(ja - G, t, rows_b, sem_sb)

            gath_start(jb, rows_b, sem_gb)
            for t in range(G):
                scat_start(ja, t, rows_a, sem_sa)
            gath_wait(jb, rows_b, sem_gb)
            for t in range(G):
                scat_wait(ja, t, rows_a, sem_sa)

            @pl.when(k + 1 < ngrp)
            def _():
                gath_start(ja + 2 * G, rows_a, sem_ga)

            for t in range(G):
                scat_start(jb, t, rows_b, sem_sb)
            return carry

        lax.fori_loop(0, ngrp, grp, 0)
        for t in range(G):
            scat_wait((ngrp - 1) * 2 * G + G, t, rows_b, sem_sb)
        plsc.subcore_barrier()

        @pl.when(c == 0)
        def _():
            pltpu.sync_copy(acc.at[pl.ds(s * RPT, RPT)],
                            out0_hbm.at[pl.ds(s * RPT, RPT)])

        @pl.when(c == 1)
        def _():
            pltpu.sync_copy(acc.at[pl.ds(s * RPT, RPT)],
                            out1_hbm.at[pl.ds(s * RPT, RPT)])

    return scatter_kernel


def _row(f):
    return pl.BlockSpec((RB, f), lambda i: (i, 0))


def _full(r, c):
    return pl.BlockSpec((r, c), lambda i: (0, 0))


def _prep_body(dp, x, w1, dinv_o, z1_o):
    ones32 = jnp.full((NW, 1), 1.0, jnp.float32)
    deg = lax.dot_general(dp[...], ones32, (((0,), (0,)), ((), ())),
                          preferred_element_type=jnp.float32)
    dinv = lax.rsqrt(deg + 1.0)
    dinv_o[...] = dinv
    z1_o[...] = dinv * jnp.dot(x[...], w1[...],
                               preferred_element_type=jnp.float32)


_prep = pl.pallas_call(
    _prep_body,
    grid=(GRID,),
    in_specs=[pl.BlockSpec((NW, RB), lambda i: (0, i)), _row(200), _full(200, 32)],
    out_specs=[_row(1), _row(32)],
    out_shape=[jax.ShapeDtypeStruct((N, 1), jnp.float32),
               jax.ShapeDtypeStruct((N_ACC, 32), jnp.float32)],
)


def _mid1_body(a0, a1, z1, dinv, b1, w2, z2_o):
    h = jnp.maximum(dinv[...] * (a0[...] + a1[...] + z1[...]) + b1[...], 0.0)
    z2_o[...] = dinv[...] * jnp.dot(h, w2[...],
                                    preferred_element_type=jnp.float32)


_mid1 = pl.pallas_call(
    _mid1_body,
    grid=(GRID,),
    in_specs=[_row(32), _row(32), _row(32), _row(1), _full(1, 32), _full(32, 16)],
    out_specs=_row(16),
    out_shape=jax.ShapeDtypeStruct((N_ACC, 16), jnp.float32),
)


def _mid2_body(a0, a1, z2, dinv, b2, z3_o):
    h = jnp.maximum(dinv[...] * (a0[...] + a1[...] + z2[...]) + b2[...], 0.0)
    z3_o[...] = dinv[...] * h


_mid2 = pl.pallas_call(
    _mid2_body,
    grid=(GRID,),
    in_specs=[_row(16), _row(16), _row(16), _row(1), _full(1, 16)],
    out_specs=_row(16),
    out_shape=jax.ShapeDtypeStruct((N_ACC, 16), jnp.float32),
)


def _final_body(a0, a1, z3, dinv, w3, b3, out_o):
    p = dinv[...] * (a0[...] + a1[...] + z3[...])
    h = jnp.dot(p, w3[...], preferred_element_type=jnp.float32) + b3[...]
    m = jnp.max(h, axis=1, keepdims=True)
    e = h - m
    out_o[...] = e - jnp.log(jnp.sum(jnp.exp(e), axis=1, keepdims=True))


_final = pl.pallas_call(
    _final_body,
    grid=(GRID,),
    in_specs=[_row(16), _row(16), _row(16), _row(1), _full(16, 200), _full(1, 200)],
    out_specs=_row(200),
    out_shape=jax.ShapeDtypeStruct((N, 200), jnp.float32),
)


def kernel(x, edge_index, W1, b1, W2, b2, W3, b3):
    e = edge_index.shape[1]
    nch = -(-(-(-e // (NW * CHUNK))) // (2 * G)) * (2 * G)
    ept = nch * CHUNK
    e_pad = ept * NW

    ei = edge_index.astype(jnp.int32)
    src = jnp.concatenate([ei[0], jnp.zeros((e_pad - e,), jnp.int32)])
    dst = jnp.concatenate([ei[1], jnp.full((e_pad - e,), N, jnp.int32)])
    src_r = src.reshape(NW, nch // G, WIDE)
    dst_r = dst.reshape(NW, nch, CHUNK)
    zdeg = jnp.zeros((DEGW,), jnp.float32)
    z16 = jnp.zeros((N_ACC, 16), jnp.float32)
    z32 = jnp.zeros((N_ACC, 32), jnp.float32)

    if e % (NW * 16) == 0:
        degp = _make_deg(e // NW, e)(ei.reshape(-1), zdeg)
    else:
        degp = _make_deg(ept, 0)(dst, zdeg)

    dinv, zr1 = _prep(degp, x, W1)

    a0, a1 = _make_scatter(32, nch)(zr1, src_r, dst_r, z32)
    zr2 = _mid1(a0, a1, zr1, dinv, b1.reshape(1, 32), W2)

    a0, a1 = _make_scatter(16, nch)(zr2, src_r, dst_r, z16)
    zr3 = _mid2(a0, a1, zr2, dinv, b2.reshape(1, 16))

    a0, a1 = _make_scatter(16, nch)(zr3, src_r, dst_r, z16)
    return _final(a0, a1, zr3, dinv, W3, b3.reshape(1, 200))


# speedup vs baseline: 1.0004x; 1.0004x over previous
import functools

import jax
import jax.numpy as jnp
from jax import lax
from jax.experimental import pallas as pl
from jax.experimental.pallas import tpu as pltpu
from jax.experimental.pallas import tpu_sc as plsc

N = 10000
NC, NS = 2, 16
NW = NC * NS
CHUNK = 128
RPT = 632
N_ACC = NS * RPT
DEG_ROWS = 640
DEG_RPT = DEG_ROWS // NS
DEG_CH = DEG_ROWS // CHUNK

RB = 1024
GRID = -(-N // RB)


def _sc_mesh():
    return plsc.VectorSubcoreMesh(core_axis_name="c", subcore_axis_name="s")


DEGW = DEG_ROWS * 16


def _make_deg(ept, base):
    steps = ept // 16

    @functools.partial(
        pl.kernel,
        out_type=jax.ShapeDtypeStruct((NW, DEGW), jnp.float32),
        mesh=_sc_mesh(),
        compiler_params=pltpu.CompilerParams(needs_layout_passes=False),
        scratch_types=[
            pltpu.VMEM((ept,), jnp.int32),
            pltpu.VMEM((DEGW,), jnp.float32),
        ],
    )
    def deg_kernel(dst1_hbm, zeros_hbm, out_hbm, idxv, degv):
        c = lax.axis_index("c")
        s = lax.axis_index("s")
        wid = c * NS + s
        pltpu.sync_copy(zeros_hbm, degv)
        pltpu.sync_copy(dst1_hbm.at[pl.ds(base + wid * ept, ept)], idxv)
        ones = jnp.full((16,), 1.0, jnp.float32)

        def step(i, carry):
            idx = idxv[pl.ds(i * 16, 16)]
            plsc.addupdate_scatter(degv, [idx], ones)
            return carry

        lax.fori_loop(0, steps, step, 0)
        pltpu.sync_copy(degv, out_hbm.at[wid])

    return deg_kernel


G = 8
WIDE = G * CHUNK


def _make_scatter(f, nch):
    ngrp = nch // (2 * G)

    @functools.partial(
        pl.kernel,
        out_type=[jax.ShapeDtypeStruct((N_ACC, f), jnp.float32),
                  jax.ShapeDtypeStruct((N_ACC, f), jnp.float32)],
        mesh=_sc_mesh(),
        compiler_params=pltpu.CompilerParams(use_tc_tiling_on_sc=False),
        scratch_types=[
            pltpu.VMEM((nch // G, WIDE), jnp.int32),
            pltpu.VMEM((nch, CHUNK), jnp.int32),
            pltpu.VMEM((WIDE, f), jnp.float32),
            pltpu.VMEM((WIDE, f), jnp.float32),
            pltpu.VMEM_SHARED((N_ACC, f), jnp.float32),
            pltpu.VMEM_SHARED((N_ACC, f), jnp.float32),
            pltpu.SemaphoreType.DMA,
            pltpu.SemaphoreType.DMA,
            pltpu.SemaphoreType.DMA,
            pltpu.SemaphoreType.DMA,
        ],
    )
    def scatter_kernel(z_hbm, src_hbm, dst_hbm, zeros_hbm, out0_hbm, out1_hbm,
                       idxs, idxd, rows_a, rows_b, acc, zs,
                       sem_ga, sem_gb, sem_sa, sem_sb):
        c = lax.axis_index("c")
        s = lax.axis_index("s")
        wid = c * NS + s
        pltpu.sync_copy(zeros_hbm.at[pl.ds(s * RPT, RPT)],
                        acc.at[pl.ds(s * RPT, RPT)])
        pltpu.sync_copy(z_hbm.at[pl.ds(s * RPT, RPT)],
                        zs.at[pl.ds(s * RPT, RPT)])
        pltpu.sync_copy(src_hbm.at[wid], idxs)
        pltpu.sync_copy(dst_hbm.at[wid], idxd)
        plsc.subcore_barrier()

        def gath_start(j, buf, sem):
            pltpu.async_copy(zs.at[idxs.at[j // G]], buf, sem)

        def gath_wait(j, buf, sem):
            pltpu.make_async_copy(zs.at[idxs.at[j // G]], buf, sem).wait()

        def scat_start(j, t, buf, sem):
            pltpu.async_copy(buf.at[pl.ds(t * CHUNK, CHUNK)],
                             acc.at[idxd.at[j + t]], sem, add=True)

        def scat_wait(j, t, buf, sem):
            pltpu.make_async_copy(buf.at[pl.ds(t * CHUNK, CHUNK)],
                                  acc.at[idxd.at[j + t]], sem).wait()

        gath_start(0, rows_a, sem_ga)

        def grp(k, carry):
            ja = k * 2 * G
            jb = ja + G
            gath_wait(ja, rows_a, sem_ga)
            gath_start(jb, rows_b, sem_gb)
            for t in range(G):
                scat_start(ja, t, rows_a, sem_sa)
            for t in range(G):
                scat_wait(ja, t, rows_a, sem_sa)
            gath_wait(jb, rows_b, sem_gb)

            @pl.when(k + 1 < ngrp)
            def _():
                gath_start(ja + 2 * G, rows_a, sem_ga)

            for t in range(G):
                scat_start(jb, t, rows_b, sem_sb)
            for t in range(G):
                scat_wait(jb, t, rows_b, sem_sb)
            return carry

        lax.fori_loop(0, ngrp, grp, 0)
        plsc.subcore_barrier()

        @pl.when(c == 0)
        def _():
            pltpu.sync_copy(acc.at[pl.ds(s * RPT, RPT)],
                            out0_hbm.at[pl.ds(s * RPT, RPT)])

        @pl.when(c == 1)
        def _():
            pltpu.sync_copy(acc.at[pl.ds(s * RPT, RPT)],
                            out1_hbm.at[pl.ds(s * RPT, RPT)])

    return scatter_kernel


def _row(f):
    return pl.BlockSpec((RB, f), lambda i: (i, 0))


def _full(r, c):
    return pl.BlockSpec((r, c), lambda i: (0, 0))


def _prep_body(dp, x, w1, dinv_o, z1_o):
    ones32 = jnp.full((NW, 1), 1.0, jnp.float32)
    deg = lax.dot_general(dp[...], ones32, (((0,), (0,)), ((), ())),
                          preferred_element_type=jnp.float32)
    dinv = lax.rsqrt(deg + 1.0)
    dinv_o[...] = dinv
    z1_o[...] = dinv * jnp.dot(x[...], w1[...],
                               preferred_element_type=jnp.float32)


_prep = pl.pallas_call(
    _prep_body,
    grid=(GRID,),
    in_specs=[pl.BlockSpec((NW, RB), lambda i: (0, i)), _row(200), _full(200, 32)],
    out_specs=[_row(1), _row(32)],
    out_shape=[jax.ShapeDtypeStruct((N, 1), jnp.float32),
               jax.ShapeDtypeStruct((N_ACC, 32), jnp.float32)],
)


def _mid1_body(a0, a1, z1, dinv, b1, w2, z2_o):
    h = jnp.maximum(dinv[...] * (a0[...] + a1[...] + z1[...]) + b1[...], 0.0)
    z2_o[...] = dinv[...] * jnp.dot(h, w2[...],
                                    preferred_element_type=jnp.float32)


_mid1 = pl.pallas_call(
    _mid1_body,
    grid=(GRID,),
    in_specs=[_row(32), _row(32), _row(32), _row(1), _full(1, 32), _full(32, 16)],
    out_specs=_row(16),
    out_shape=jax.ShapeDtypeStruct((N_ACC, 16), jnp.float32),
)


def _mid2_body(a0, a1, z2, dinv, b2, z3_o):
    h = jnp.maximum(dinv[...] * (a0[...] + a1[...] + z2[...]) + b2[...], 0.0)
    z3_o[...] = dinv[...] * h


_mid2 = pl.pallas_call(
    _mid2_body,
    grid=(GRID,),
    in_specs=[_row(16), _row(16), _row(16), _row(1), _full(1, 16)],
    out_specs=_row(16),
    out_shape=jax.ShapeDtypeStruct((N_ACC, 16), jnp.float32),
)


def _final_body(a0, a1, z3, dinv, w3, b3, out_o):
    p = dinv[...] * (a0[...] + a1[...] + z3[...])
    h = jnp.dot(p, w3[...], preferred_element_type=jnp.float32) + b3[...]
    m = jnp.max(h, axis=1, keepdims=True)
    e = h - m
    out_o[...] = e - jnp.log(jnp.sum(jnp.exp(e), axis=1, keepdims=True))


_final = pl.pallas_call(
    _final_body,
    grid=(GRID,),
    in_specs=[_row(16), _row(16), _row(16), _row(1), _full(16, 200), _full(1, 200)],
    out_specs=_row(200),
    out_shape=jax.ShapeDtypeStruct((N, 200), jnp.float32),
)


def kernel(x, edge_index, W1, b1, W2, b2, W3, b3):
    e = edge_index.shape[1]
    nch = -(-(-(-e // (NW * CHUNK))) // (2 * G)) * (2 * G)
    ept = nch * CHUNK
    e_pad = ept * NW

    ei = edge_index.astype(jnp.int32)
    src = jnp.concatenate([ei[0], jnp.zeros((e_pad - e,), jnp.int32)])
    dst = jnp.concatenate([ei[1], jnp.full((e_pad - e,), N, jnp.int32)])
    src_r = src.reshape(NW, nch // G, WIDE)
    dst_r = dst.reshape(NW, nch, CHUNK)
    zdeg = jnp.zeros((DEGW,), jnp.float32)
    z16 = jnp.zeros((N_ACC, 16), jnp.float32)
    z32 = jnp.zeros((N_ACC, 32), jnp.float32)

    if e % (NW * 16) == 0:
        degp = _make_deg(e // NW, e)(ei.reshape(-1), zdeg)
    else:
        degp = _make_deg(ept, 0)(dst, zdeg)

    dinv, zr1 = _prep(degp, x, W1)

    a0, a1 = _make_scatter(32, nch)(zr1, src_r, dst_r, z32)
    zr2 = _mid1(a0, a1, zr1, dinv, b1.reshape(1, 32), W2)

    a0, a1 = _make_scatter(16, nch)(zr2, src_r, dst_r, z16)
    zr3 = _mid2(a0, a1, zr2, dinv, b2.reshape(1, 16))

    a0, a1 = _make_scatter(16, nch)(zr3, src_r, dst_r, z16)
    return _final(a0, a1, zr3, dinv, W3, b3.reshape(1, 200))
